# baseline (speedup 1.0000x reference)
import functools

import jax
import jax.numpy as jnp
from jax import lax
from jax.experimental import pallas as pl
from jax.experimental.pallas import tpu as pltpu
from jax.experimental.pallas import tpu_sc as plsc


def _tc_scores_body(xq_ref, ek_ref, idx_out):
    xq = xq_ref[...]
    ek = ek_ref[...]
    nk = ek / jnp.maximum(
        jnp.sqrt(jnp.sum(ek * ek, axis=1, keepdims=True)), 1e-12)
    nq = xq / jnp.maximum(
        jnp.sqrt(jnp.sum(xq * xq, axis=1, keepdims=True)), 1e-12)
    scores = jax.lax.dot_general(nq, nk, (((1,), (1,)), ((), ())))
    idx_out[...] = jnp.argmax(scores, axis=1).astype(jnp.int32)


def _tc_scores(x_querry, e_k, blk=512):
    b, key_d = x_querry.shape
    pool, _ = e_k.shape
    grid = (b // blk,)
    return pl.pallas_call(
        _tc_scores_body,
        grid=grid,
        in_specs=[
            pl.BlockSpec((blk, key_d), lambda i: (i, 0)),
            pl.BlockSpec((pool, key_d), lambda i: (0, 0)),
        ],
        out_specs=[
            pl.BlockSpec((blk,), lambda i: (i,)),
        ],
        out_shape=[
            jax.ShapeDtypeStruct((b,), jnp.int32),
        ],
    )(x_querry, e_k)


def _tc_copy_body(xb_ref, xb_out):
    xb_out[...] = xb_ref[...]


def _tc_copy(x_block, blk=512):
    b, emb_d = x_block.shape
    return pl.pallas_call(
        _tc_copy_body,
        grid=(b // blk,),
        in_specs=[pl.BlockSpec((blk, emb_d), lambda i: (i, 0))],
        out_specs=[pl.BlockSpec((blk, emb_d), lambda i: (i, 0))],
        out_shape=[jax.ShapeDtypeStruct((b, emb_d), jnp.float32)],
    )(x_block)[0]


def _sc_gather(e_p, idx, nc, ns, ch=8):
    b = idx.shape[0]
    pool, p_len, emb_d = e_p.shape
    half = p_len // 2
    nw = nc * ns
    bw = b // nw
    nch = bw // ch
    mesh = plsc.VectorSubcoreMesh(core_axis_name="c", subcore_axis_name="s")

    nb = 4
    groups = (2 * nch) // nb

    @functools.partial(
        pl.kernel,
        out_type=[
            jax.ShapeDtypeStruct((b, half, emb_d), jnp.float32),
            jax.ShapeDtypeStruct((b, half, emb_d), jnp.float32),
        ],
        mesh=mesh,
        scratch_types=[
            pltpu.VMEM((bw,), jnp.int32),
            pltpu.VMEM((nb, ch, half, emb_d), jnp.float32),
            pltpu.SemaphoreType.DMA,
            pltpu.SemaphoreType.DMA,
            pltpu.SemaphoreType.DMA,
            pltpu.SemaphoreType.DMA,
            pltpu.SemaphoreType.DMA,
            pltpu.SemaphoreType.DMA,
            pltpu.SemaphoreType.DMA,
            pltpu.SemaphoreType.DMA,
        ],
    )
    def k(ep_hbm, idx_hbm, ek_out, ev_out,
          idx_v, bufs, g0, g1, g2, g3, o0, o1, o2, o3):
        wid = lax.axis_index("s") * nc + lax.axis_index("c")
        base = wid * bw

        pltpu.sync_copy(idx_hbm.at[pl.ds(base, bw)], idx_v)

        gsem = [g0, g1, g2, g3]
        osem = [o0, o1, o2, o3]

        def job_refs(g, r):
            c = 2 * g + r // 2
            h = r % 2
            off = pl.multiple_of(c * ch, 8)
            iref = idx_v.at[pl.ds(off, ch)]
            src = ep_hbm.at[iref, pl.ds(h * half, half)]
            dst = (ek_out if h == 0 else ev_out).at[pl.ds(base + c * ch, ch)]
            return src, dst

        def fire_gather(g, r):
            src, _ = job_refs(g, r)
            pltpu.async_copy(src, bufs.at[r], gsem[r])

        def process(g, r, refill):
            src, dst = job_refs(g, r)
            pltpu.make_async_copy(src, bufs.at[r], gsem[r]).wait()
            pltpu.async_copy(bufs.at[r], dst, osem[r])
            if refill:
                pltpu.make_async_copy(bufs.at[r], dst, osem[r]).wait()
                fire_gather(g + 1, r)

        for r in range(nb):
            fire_gather(0, r)

        def body(g, carry):
            for r in range(nb):
                process(g, r, True)
            return carry

        lax.fori_loop(0, groups - 1, body, 0)
        for r in range(nb):
            process(groups - 1, r, False)
            pltpu.make_async_copy(
                bufs.at[r], job_refs(groups - 1, r)[1], osem[r]).wait()

    return k(e_p, idx)


def kernel(x_querry, l, x_block, e_p, e_k):
    b = x_querry.shape[0]

    (idx,) = _tc_scores(x_querry, e_k)

    info = plsc.get_sparse_core_info()
    ekf, evf = _sc_gather(e_p, idx, info.num_cores, info.num_subcores)
    xb_out = _tc_copy(x_block)
    return (ekf, evf, xb_out)


# speedup vs baseline: 1.0091x; 1.0091x over previous
import functools

import jax
import jax.numpy as jnp
from jax import lax
from jax.experimental import pallas as pl
from jax.experimental.pallas import tpu as pltpu
from jax.experimental.pallas import tpu_sc as plsc


def _tc_scores_body(xq_ref, ek_ref, xb_ref, idx_out, xb_out):
    xq = xq_ref[...]
    ek = ek_ref[...]
    nk = ek / jnp.maximum(
        jnp.sqrt(jnp.sum(ek * ek, axis=1, keepdims=True)), 1e-12)
    nq = xq / jnp.maximum(
        jnp.sqrt(jnp.sum(xq * xq, axis=1, keepdims=True)), 1e-12)
    scores = jax.lax.dot_general(nq, nk, (((1,), (1,)), ((), ())))
    idx_out[...] = jnp.argmax(scores, axis=1).astype(jnp.int32)
    xb_out[...] = xb_ref[...]


def _tc_scores(x_querry, x_block, e_k, blk=512):
    b, key_d = x_querry.shape
    pool, _ = e_k.shape
    emb_d = x_block.shape[1]
    grid = (b // blk,)
    return pl.pallas_call(
        _tc_scores_body,
        grid=grid,
        in_specs=[
            pl.BlockSpec((blk, key_d), lambda i: (i, 0)),
            pl.BlockSpec((pool, key_d), lambda i: (0, 0)),
            pl.BlockSpec((blk, emb_d), lambda i: (i, 0)),
        ],
        out_specs=[
            pl.BlockSpec((blk,), lambda i: (i,)),
            pl.BlockSpec((blk, emb_d), lambda i: (i, 0)),
        ],
        out_shape=[
            jax.ShapeDtypeStruct((b,), jnp.int32),
            jax.ShapeDtypeStruct((b, emb_d), jnp.float32),
        ],
    )(x_querry, e_k, x_block)


def _sc_gather(e_p, idx, nc, ns, ch=8):
    b = idx.shape[0]
    pool, p_len, emb_d = e_p.shape
    half = p_len // 2
    nw = nc * ns
    bw = b // nw
    nch = bw // ch
    mesh = plsc.VectorSubcoreMesh(core_axis_name="c", subcore_axis_name="s")

    nb = 5

    @functools.partial(
        pl.kernel,
        out_type=[
            jax.ShapeDtypeStruct((b, half, emb_d), jnp.float32),
            jax.ShapeDtypeStruct((b, half, emb_d), jnp.float32),
        ],
        mesh=mesh,
        scratch_types=[
            pltpu.VMEM((bw,), jnp.int32),
            pltpu.VMEM((nb, ch, half, emb_d), jnp.float32),
            pltpu.SemaphoreType.DMA,
            pltpu.SemaphoreType.DMA,
            pltpu.SemaphoreType.DMA,
            pltpu.SemaphoreType.DMA,
            pltpu.SemaphoreType.DMA,
            pltpu.SemaphoreType.DMA,
            pltpu.SemaphoreType.DMA,
            pltpu.SemaphoreType.DMA,
            pltpu.SemaphoreType.DMA,
            pltpu.SemaphoreType.DMA,
        ],
    )
    def k(ep_hbm, idx_hbm, ek_out, ev_out,
          idx_v, bufs, g0, g1, g2, g3, g4, o0, o1, o2, o3, o4):
        wid = lax.axis_index("s") * nc + lax.axis_index("c")
        base = wid * bw

        pltpu.sync_copy(idx_hbm.at[pl.ds(base, bw)], idx_v)

        gsem = [g0, g1, g2, g3, g4]
        osem = [o0, o1, o2, o3, o4]
        jobs = 2 * nch

        def fire(j):
            c, h = divmod(j, 2)
            iref = idx_v.at[pl.ds(c * ch, ch)]
            src = ep_hbm.at[iref, pl.ds(h * half, half)]
            return pltpu.async_copy(src, bufs.at[j % nb], gsem[j % nb])

        gh = [None] * jobs
        oh = [None] * jobs
        for j in range(min(nb, jobs)):
            gh[j] = fire(j)
        for j in range(jobs):
            r = j % nb
            gh[j].wait()
            c, h = divmod(j, 2)
            dst = (ek_out if h == 0 else ev_out).at[pl.ds(base + c * ch, ch)]
            oh[j] = pltpu.async_copy(bufs.at[r], dst, osem[r])
            if j + nb < jobs:
                oh[j].wait()
                gh[j + nb] = fire(j + nb)
        for j in range(max(0, jobs - nb), jobs):
            oh[j].wait()

    return k(e_p, idx)


def kernel(x_querry, l, x_block, e_p, e_k):
    idx, xb_out = _tc_scores(x_querry, x_block, e_k)

    info = plsc.get_sparse_core_info()
    ekf, evf = _sc_gather(e_p, idx, info.num_cores, info.num_subcores)
    return (ekf, evf, xb_out)
